# baseline (speedup 1.0000x reference)
import functools

import jax
import jax.numpy as jnp
from jax.experimental import pallas as pl


def _emd_reduce_kernel(p_ref, t_ref, o_ref, *, c, inv_n, inv_b):
    p = p_ref[:]
    t = t_ref[:]
    total = jnp.sum(p * p + t * t, keepdims=True)
    idx = jax.lax.broadcasted_iota(jnp.int32, p.shape, 1) % c
    cross = jnp.zeros((1, 1), dtype=jnp.float32)
    for cc in range(c):
        m = idx == cc
        spc = jnp.sum(jnp.where(m, p, 0.0), axis=1, keepdims=True)
        stc = jnp.sum(jnp.where(m, t, 0.0), axis=1, keepdims=True)
        cross = cross + jnp.sum(spc * stc, keepdims=True)
    o_ref[:, :] = (total - 2.0 * inv_n * cross) * inv_b


def kernel(pred, target):
    b, n, c = pred.shape
    p = pred.reshape(b, n * c)
    t = target.reshape(b, n * c)
    out = pl.pallas_call(
        functools.partial(_emd_reduce_kernel, c=c, inv_n=1.0 / n, inv_b=1.0 / b),
        out_shape=jax.ShapeDtypeStruct((1, 1), jnp.float32),
    )(p, t)
    return out[0, 0]
